# Initial kernel scaffold; baseline (speedup 1.0000x reference)
import jax
import jax.numpy as jnp
from jax.experimental import pallas as pl


def kernel(decoder_outputs, attention_scores, input_sequence, repeat_idx, repeat_idx2, convert_table):
    raise NotImplementedError("write your pallas kernel here")


# speedup vs baseline: 5.5817x; 5.5817x over previous
import functools

import jax
import jax.numpy as jnp
from jax import lax
from jax.experimental import pallas as pl
from jax.experimental.pallas import tpu as pltpu
from jax.experimental.pallas import tpu_sc as plsc

B = 16
T = 64
V = 30000
A = 256
L = 16


def _sc_body(dec_hbm, attn_hbm, iseq_hbm, ct_hbm, out_hbm,
             iseq_v, ids_v, colmax_v, attn_v, row_v, sem):
    core = lax.axis_index("c")
    sub = lax.axis_index("s")
    b = sub
    t0 = core * (T // 2)
    iota = lax.iota(jnp.int32, L)

    pltpu.sync_copy(iseq_hbm.at[b], iseq_v)
    for j in range(2):
        pltpu.async_copy(ct_hbm.at[iseq_v.at[j]],
                         ids_v.at[pl.ds(j * 128, 128)], sem).wait()

    pltpu.sync_copy(attn_hbm.at[b], attn_v)
    for c in range(A // L):
        colmax_v[pl.ds(c * L, L)] = attn_v[0, pl.ds(c * L, L)]

    def _red_body(tt, _):
        for c in range(A // L):
            sl = pl.ds(c * L, L)
            colmax_v[sl] = jnp.maximum(colmax_v[sl], attn_v[tt, sl])
        return ()

    lax.fori_loop(1, T, _red_body, (), unroll=False)

    def _row_body(t, _):
        pltpu.sync_copy(dec_hbm.at[b, t], row_v)

        idx4 = t + jnp.minimum(iota, 3) * T
        ids4 = plsc.load_gather(ids_v, [idx4])
        vals4 = plsc.load_gather(colmax_v, [idx4])
        for k in range(A // T):
            old = plsc.load_gather(row_v, [ids4])
            new = jnp.maximum(old, vals4)
            plsc.store_scatter(row_v, [ids4], new, mask=(iota == k))

        head = row_v[pl.ds(0, L)]
        row_v[pl.ds(0, L)] = jnp.where(iota == 1, jnp.float32(0.0), head)

        pltpu.sync_copy(row_v, out_hbm.at[b, t])
        return ()

    lax.fori_loop(t0, t0 + T // 2, _row_body, (), unroll=False)


@functools.partial(jax.jit, static_argnames=())
def _pointer_generator_sc(dec, attn, iseq2, ct):
    mesh = plsc.VectorSubcoreMesh(core_axis_name="c", subcore_axis_name="s")
    return pl.kernel(
        _sc_body,
        out_type=jax.ShapeDtypeStruct((B, T, V), jnp.float32),
        mesh=mesh,
        compiler_params=pltpu.CompilerParams(needs_layout_passes=False),
        scratch_types=[
            pltpu.VMEM((2, 128), jnp.int32),
            pltpu.VMEM((A,), jnp.int32),
            pltpu.VMEM((A,), jnp.float32),
            pltpu.VMEM((T, A), jnp.float32),
            pltpu.VMEM((V,), jnp.float32),
            pltpu.SemaphoreType.DMA,
        ],
    )(dec, attn, iseq2, ct)


def kernel(decoder_outputs, attention_scores, input_sequence,
           repeat_idx, repeat_idx2, convert_table):
    del repeat_idx, repeat_idx2
    iseq2 = input_sequence.reshape(B, 2, 128)
    return _pointer_generator_sc(decoder_outputs, attention_scores,
                                 iseq2, convert_table)


# speedup vs baseline: 6.8312x; 1.2239x over previous
import functools

import jax
import jax.numpy as jnp
from jax import lax
from jax.experimental import pallas as pl
from jax.experimental.pallas import tpu as pltpu
from jax.experimental.pallas import tpu_sc as plsc

B = 16
T = 64
V = 30000
A = 256
L = 16
NB = 3
ROWS = T // 2


def _sc_body(dec_hbm, attn_hbm, iseq_hbm, ct_hbm, out_hbm,
             iseq_v, ids_v, colmax_v, attn_v, row0, row1, row2,
             gsem, asem, in_sems, out_sems):
    core = lax.axis_index("c")
    sub = lax.axis_index("s")
    b = sub
    t0 = core * ROWS
    iota = lax.iota(jnp.int32, L)
    rows = (row0, row1, row2)

    attn_cp = pltpu.async_copy(attn_hbm.at[b], attn_v, asem)
    in_descs = {}
    out_descs = {}
    for i in range(NB - 1):
        in_descs[i] = pltpu.async_copy(dec_hbm.at[b, t0 + i], rows[i],
                                       in_sems[i])

    pltpu.sync_copy(iseq_hbm.at[b], iseq_v)
    for j in range(2):
        pltpu.async_copy(ct_hbm.at[iseq_v.at[j]],
                         ids_v.at[pl.ds(j * 128, 128)], gsem).wait()

    attn_cp.wait()
    for c in range(A // L):
        colmax_v[pl.ds(c * L, L)] = attn_v[0, pl.ds(c * L, L)]

    def _red_body(tt, _):
        for c in range(A // L):
            sl = pl.ds(c * L, L)
            colmax_v[sl] = jnp.maximum(colmax_v[sl], attn_v[tt, sl])
        return ()

    lax.fori_loop(1, T, _red_body, (), unroll=False)

    for j in range(ROWS):
        buf = j % NB
        pj = j + NB - 1
        if pj < ROWS:
            pbuf = pj % NB
            if pj - NB >= 0:
                out_descs[pj - NB].wait()
            in_descs[pj] = pltpu.async_copy(dec_hbm.at[b, t0 + pj],
                                            rows[pbuf], in_sems[pbuf])
        in_descs[j].wait()
        row_v = rows[buf]
        t = t0 + j

        idx4 = t + jnp.minimum(iota, 3) * T
        ids4 = plsc.load_gather(ids_v, [idx4])
        vals4 = plsc.load_gather(colmax_v, [idx4])
        for k in range(A // T):
            old = plsc.load_gather(row_v, [ids4])
            new = jnp.maximum(old, vals4)
            plsc.store_scatter(row_v, [ids4], new, mask=(iota == k))

        head = row_v[pl.ds(0, L)]
        row_v[pl.ds(0, L)] = jnp.where(iota == 1, jnp.float32(0.0), head)

        out_descs[j] = pltpu.async_copy(row_v, out_hbm.at[b, t],
                                        out_sems[buf])
    for j in range(ROWS - NB, ROWS):
        out_descs[j].wait()


@jax.jit
def _pointer_generator_sc(dec, attn, iseq2, ct):
    mesh = plsc.VectorSubcoreMesh(core_axis_name="c", subcore_axis_name="s")
    return pl.kernel(
        _sc_body,
        out_type=jax.ShapeDtypeStruct((B, T, V), jnp.float32),
        mesh=mesh,
        compiler_params=pltpu.CompilerParams(needs_layout_passes=False),
        scratch_types=[
            pltpu.VMEM((2, 128), jnp.int32),
            pltpu.VMEM((A,), jnp.int32),
            pltpu.VMEM((A,), jnp.float32),
            pltpu.VMEM((T, A), jnp.float32),
            pltpu.VMEM((V,), jnp.float32),
            pltpu.VMEM((V,), jnp.float32),
            pltpu.VMEM((V,), jnp.float32),
            pltpu.SemaphoreType.DMA,
            pltpu.SemaphoreType.DMA,
            [pltpu.SemaphoreType.DMA] * NB,
            [pltpu.SemaphoreType.DMA] * NB,
        ],
    )(dec, attn, iseq2, ct)


def kernel(decoder_outputs, attention_scores, input_sequence,
           repeat_idx, repeat_idx2, convert_table):
    del repeat_idx, repeat_idx2
    iseq2 = input_sequence.reshape(B, 2, 128)
    return _pointer_generator_sc(decoder_outputs, attention_scores,
                                 iseq2, convert_table)
